# baseline (speedup 1.0000x reference)
import functools

import jax
import jax.numpy as jnp
from jax import lax
from jax.experimental import pallas as pl
from jax.experimental.pallas import tpu as pltpu, tpu_sc as plsc

N = 10000
E = 320000
D = 128
EDGE_DROP = 0.2
FEAT_MASK = 0.2

NP = 10240
ZR = N
BS = 512
NBR = NP // BS

NC = 2
NS = 16
CHUNK = 128
G = 8
NG = 20
NCHUNK = G * NG
EPT = NCHUNK * CHUNK
EPAD = EPT * NS - E
RPT = NP // NS


def _prep_edges(src, dst):
    rkey = jax.random.key(42)
    k1, k2, k3, k4 = jax.random.split(rkey, 4)
    em1 = jax.random.bernoulli(k1, 1.0 - EDGE_DROP, (E,))
    em2 = jax.random.bernoulli(k2, 1.0 - EDGE_DROP, (E,))
    fm1 = jax.random.bernoulli(k3, 1.0 - FEAT_MASK, (D,)).astype(jnp.float32)
    fm2 = jax.random.bernoulli(k4, 1.0 - FEAT_MASK, (D,)).astype(jnp.float32)

    s1 = jnp.where(em1, src, ZR)
    s2 = jnp.where(em2, src, ZR) + NP
    padi = jnp.full((EPAD,), ZR, jnp.int32)
    s1 = jnp.concatenate([s1, padi])
    s2 = jnp.concatenate([s2, padi + NP])
    dstp = jnp.concatenate([dst, jnp.zeros((EPAD,), jnp.int32)])
    src_t = jnp.stack([s1, s2]).reshape(NC, NS, NCHUNK, CHUNK)
    dst_t = jnp.stack([dstp, dstp]).reshape(NC, NS, NCHUNK, CHUNK)
    return src_t, dst_t, fm1, fm2


def _sc_agg(y, src_t, dst_t, zrows):
    mesh = plsc.VectorSubcoreMesh(core_axis_name="c", subcore_axis_name="s")

    @functools.partial(
        pl.kernel,
        out_type=jax.ShapeDtypeStruct((NC * NP, D), jnp.float32),
        mesh=mesh,
        scratch_types=[
            pltpu.VMEM((G, CHUNK), jnp.int32),
            pltpu.VMEM((G, CHUNK), jnp.int32),
            pltpu.VMEM((CHUNK, D), jnp.float32),
            pltpu.VMEM_SHARED((NP, D), jnp.float32),
            pltpu.SemaphoreType.DMA,
        ],
    )
    def k(y_hbm, src_hbm, dst_hbm, z_hbm, agg_hbm, sidx, didx, rows, acc, sem):
        c = lax.axis_index("c")
        s = lax.axis_index("s")
        r0 = s * RPT
        pltpu.sync_copy(z_hbm.at[pl.ds(r0, RPT)], acc.at[pl.ds(r0, RPT)])
        plsc.subcore_barrier()

        def group(g, carry):
            pltpu.sync_copy(src_hbm.at[c, s, pl.ds(g * G, G)], sidx)
            pltpu.sync_copy(dst_hbm.at[c, s, pl.ds(g * G, G)], didx)
            for j in range(G):
                pltpu.async_copy(y_hbm.at[sidx.at[j]], rows, sem).wait()
                pltpu.sync_copy(rows, acc.at[didx.at[j]], add=True)
            return carry

        lax.fori_loop(0, NG, group, 0)
        plsc.subcore_barrier()
        pltpu.sync_copy(acc.at[pl.ds(r0, RPT)],
                        agg_hbm.at[pl.ds(c * NP + r0, RPT)])

    return k(y, src_t, dst_t, zrows)


def _tc_first_body(x_ref, fm_ref, w_ref, o_ref):
    o_ref[...] = jnp.dot(x_ref[...] * fm_ref[0], w_ref[...],
                         preferred_element_type=jnp.float32)


def _tc_first(x0, fms, w_rel0):
    return pl.pallas_call(
        _tc_first_body,
        grid=(2, NBR),
        in_specs=[
            pl.BlockSpec((BS, D), lambda v, i: (v * NBR + i, 0)),
            pl.BlockSpec((1, 1, D), lambda v, i: (v, 0, 0)),
            pl.BlockSpec((D, D), lambda v, i: (0, 0)),
        ],
        out_specs=pl.BlockSpec((BS, D), lambda v, i: (v * NBR + i, 0)),
        out_shape=jax.ShapeDtypeStruct((2 * NP, D), jnp.float32),
    )(x0, fms.reshape(2, 1, D), w_rel0)


def _tc_mid_body(agg_ref, x_ref, fm_ref, wroot_ref, b_ref, wnext_ref,
                 xn_ref, yn_ref):
    t = (agg_ref[...]
         + jnp.dot(x_ref[...] * fm_ref[0], wroot_ref[...],
                   preferred_element_type=jnp.float32)
         + b_ref[...])
    t = jnp.maximum(t, 0.0)
    rows = pl.program_id(1) * BS + lax.broadcasted_iota(jnp.int32, (BS, D), 0)
    t = jnp.where(rows < N, t, 0.0)
    xn_ref[...] = t
    yn_ref[...] = jnp.dot(t, wnext_ref[...], preferred_element_type=jnp.float32)


def _tc_mid(agg, x, fms, w_root, b, w_rel_next):
    return pl.pallas_call(
        _tc_mid_body,
        grid=(2, NBR),
        in_specs=[
            pl.BlockSpec((BS, D), lambda v, i: (v * NBR + i, 0)),
            pl.BlockSpec((BS, D), lambda v, i: (v * NBR + i, 0)),
            pl.BlockSpec((1, 1, D), lambda v, i: (v, 0, 0)),
            pl.BlockSpec((D, D), lambda v, i: (0, 0)),
            pl.BlockSpec((1, D), lambda v, i: (0, 0)),
            pl.BlockSpec((D, D), lambda v, i: (0, 0)),
        ],
        out_specs=[
            pl.BlockSpec((BS, D), lambda v, i: (v * NBR + i, 0)),
            pl.BlockSpec((BS, D), lambda v, i: (v * NBR + i, 0)),
        ],
        out_shape=[
            jax.ShapeDtypeStruct((2 * NP, D), jnp.float32),
            jax.ShapeDtypeStruct((2 * NP, D), jnp.float32),
        ],
    )(agg, x, fms.reshape(2, 1, D), w_root, b.reshape(1, D), w_rel_next)


def _tc_last_body(agg_ref, x_ref, wroot_ref, b_ref, wlin_ref, blin_ref, z_ref):
    t = (agg_ref[...]
         + jnp.dot(x_ref[...], wroot_ref[...], preferred_element_type=jnp.float32)
         + b_ref[...])
    t = jnp.maximum(t, 0.0)
    z_ref[...] = (jnp.dot(t, wlin_ref[...], preferred_element_type=jnp.float32)
                  + blin_ref[...])


def _tc_last(agg, x, w_root, b, w_lin, b_lin):
    return pl.pallas_call(
        _tc_last_body,
        grid=(2, NBR),
        in_specs=[
            pl.BlockSpec((BS, D), lambda v, i: (v * NBR + i, 0)),
            pl.BlockSpec((BS, D), lambda v, i: (v * NBR + i, 0)),
            pl.BlockSpec((D, D), lambda v, i: (0, 0)),
            pl.BlockSpec((1, D), lambda v, i: (0, 0)),
            pl.BlockSpec((D, D), lambda v, i: (0, 0)),
            pl.BlockSpec((1, D), lambda v, i: (0, 0)),
        ],
        out_specs=pl.BlockSpec((BS, D), lambda v, i: (v * NBR + i, 0)),
        out_shape=jax.ShapeDtypeStruct((2 * NP, D), jnp.float32),
    )(agg, x, w_root, b.reshape(1, D), w_lin, b_lin.reshape(1, D))


def kernel(x, edge_index, W_rel0, b_rel0, W_root0, W_rel1, b_rel1, W_root1,
           W_rel2, b_rel2, W_root2, W_lin, b_lin):
    src = edge_index[0]
    dst = edge_index[1]
    src_t, dst_t, fm1, fm2 = _prep_edges(src, dst)
    fms = jnp.stack([fm1, fm2])
    ones = jnp.ones_like(fms)
    xp = jnp.zeros((NP, D), jnp.float32).at[:N].set(x)
    x0 = jnp.concatenate([xp, xp], axis=0)
    zrows = jnp.zeros((NP, D), jnp.float32)

    y0 = _tc_first(x0, fms, W_rel0)
    a0 = _sc_agg(y0, src_t, dst_t, zrows)
    x1, y1 = _tc_mid(a0, x0, fms, W_root0, b_rel0, W_rel1)
    a1 = _sc_agg(y1, src_t, dst_t, zrows)
    x2, y2 = _tc_mid(a1, x1, ones, W_root1, b_rel1, W_rel2)
    a2 = _sc_agg(y2, src_t, dst_t, zrows)
    z = _tc_last(a2, x2, W_root2, b_rel2, W_lin, b_lin)
    return (z[:N], z[NP:NP + N])


# speedup vs baseline: 1.0187x; 1.0187x over previous
import functools

import jax
import jax.numpy as jnp
from jax import lax
from jax.experimental import pallas as pl
from jax.experimental.pallas import tpu as pltpu, tpu_sc as plsc

N = 10000
E = 320000
D = 128
EDGE_DROP = 0.2
FEAT_MASK = 0.2

NP = 10240
ZR = N
BS = 512
NBR = NP // BS

NC = 2
NS = 16
CHUNK = 128
NBUF = 2
GC = 16
NGRP = 10
NCHUNK = GC * NGRP
RPG = GC // NBUF
NRND = NCHUNK // NBUF
EPT = NCHUNK * CHUNK
EPAD = EPT * NS - E
RPT = NP // NS


def _prep_edges(src, dst):
    rkey = jax.random.key(42)
    k1, k2, k3, k4 = jax.random.split(rkey, 4)
    em1 = jax.random.bernoulli(k1, 1.0 - EDGE_DROP, (E,))
    em2 = jax.random.bernoulli(k2, 1.0 - EDGE_DROP, (E,))
    fm1 = jax.random.bernoulli(k3, 1.0 - FEAT_MASK, (D,)).astype(jnp.float32)
    fm2 = jax.random.bernoulli(k4, 1.0 - FEAT_MASK, (D,)).astype(jnp.float32)

    s1 = jnp.where(em1, src, ZR)
    s2 = jnp.where(em2, src, ZR) + NP
    padi = jnp.full((EPAD,), ZR, jnp.int32)
    s1 = jnp.concatenate([s1, padi])
    s2 = jnp.concatenate([s2, padi + NP])
    dstp = jnp.concatenate([dst, jnp.zeros((EPAD,), jnp.int32)])
    src_r = jnp.stack([s1, s2]).reshape(NC, NS, NGRP, GC, CHUNK)
    dst_r = jnp.stack([dstp, dstp]).reshape(NC, NS, NGRP, GC, CHUNK)
    idx = jnp.stack([src_r, dst_r], axis=3)
    return idx, fm1, fm2


def _sc_agg(y, idx, zrows):
    mesh = plsc.VectorSubcoreMesh(core_axis_name="c", subcore_axis_name="s")

    @functools.partial(
        pl.kernel,
        out_type=jax.ShapeDtypeStruct((NC * NP, D), jnp.float32),
        mesh=mesh,
        scratch_types=[
            pltpu.VMEM((2, 2, GC, CHUNK), jnp.int32),
            pltpu.VMEM((CHUNK, D), jnp.float32),
            pltpu.VMEM((CHUNK, D), jnp.float32),
            pltpu.VMEM_SHARED((NP, D), jnp.float32),
            pltpu.SemaphoreType.DMA,
            pltpu.SemaphoreType.DMA,
            pltpu.SemaphoreType.DMA,
            pltpu.SemaphoreType.DMA,
            pltpu.SemaphoreType.DMA,
        ],
    )
    def k(y_hbm, idx_hbm, z_hbm, agg_hbm, ibuf, rows0, rows1, acc,
          gsem0, gsem1, ssem0, ssem1, isem):
        c = lax.axis_index("c")
        s = lax.axis_index("s")
        rows = [rows0, rows1]
        gsem = [gsem0, gsem1]
        ssem = [ssem0, ssem1]
        r0 = s * RPT
        pltpu.sync_copy(z_hbm.at[pl.ds(r0, RPT)], acc.at[pl.ds(r0, RPT)])
        pltpu.sync_copy(idx_hbm.at[c, s, 0], ibuf.at[0])
        plsc.subcore_barrier()
        for b in range(NBUF):
            pltpu.async_copy(y_hbm.at[ibuf.at[0, 0, b]], rows[b], gsem[b])

        def rnd(r, carry):
            g = r // RPG
            rr = lax.rem(r, RPG)
            slot = lax.rem(g, 2)
            @pl.when((rr == RPG - 1) & (g + 1 < NGRP))
            def _():
                pltpu.make_async_copy(idx_hbm.at[c, s, g + 1],
                                      ibuf.at[1 - slot], isem).wait()
            for b in range(NBUF):
                j = r * NBUF + b
                row = rr * NBUF + b
                pltpu.make_async_copy(y_hbm.at[ibuf.at[slot, 0, row]],
                                      rows[b], gsem[b]).wait()
                pltpu.async_copy(rows[b], acc.at[ibuf.at[slot, 1, row]],
                                 ssem[b], add=True)
                bp = (b - 1) % NBUF
                jn = j + NBUF - 1
                @pl.when((j >= 1) & (jn < NCHUNK))
                def _():
                    pltpu.make_async_copy(rows[bp],
                                          acc.at[ibuf.at[slot, 1, row]],
                                          ssem[bp]).wait()
                    gn = jn // GC
                    rown = jn - gn * GC
                    slotn = lax.rem(gn, 2)
                    pltpu.async_copy(y_hbm.at[ibuf.at[slotn, 0, rown]],
                                     rows[bp], gsem[bp])
            @pl.when((rr == 0) & (g + 1 < NGRP))
            def _():
                pltpu.async_copy(idx_hbm.at[c, s, g + 1], ibuf.at[1 - slot], isem)
            return carry

        lax.fori_loop(0, NRND, rnd, 0)
        for b in range(NBUF):
            pltpu.make_async_copy(rows[b], acc.at[ibuf.at[0, 1, 0]],
                                  ssem[b]).wait()
        plsc.subcore_barrier()
        pltpu.sync_copy(acc.at[pl.ds(r0, RPT)],
                        agg_hbm.at[pl.ds(c * NP + r0, RPT)])

    return k(y, idx, zrows)


def _tc_first_body(x_ref, fm_ref, w_ref, o_ref):
    o_ref[...] = jnp.dot(x_ref[...] * fm_ref[0], w_ref[...],
                         preferred_element_type=jnp.float32)


def _tc_first(x0, fms, w_rel0):
    return pl.pallas_call(
        _tc_first_body,
        grid=(2, NBR),
        in_specs=[
            pl.BlockSpec((BS, D), lambda v, i: (v * NBR + i, 0)),
            pl.BlockSpec((1, 1, D), lambda v, i: (v, 0, 0)),
            pl.BlockSpec((D, D), lambda v, i: (0, 0)),
        ],
        out_specs=pl.BlockSpec((BS, D), lambda v, i: (v * NBR + i, 0)),
        out_shape=jax.ShapeDtypeStruct((2 * NP, D), jnp.float32),
    )(x0, fms.reshape(2, 1, D), w_rel0)


def _tc_mid_body(agg_ref, x_ref, fm_ref, wroot_ref, b_ref, wnext_ref,
                 xn_ref, yn_ref):
    t = (agg_ref[...]
         + jnp.dot(x_ref[...] * fm_ref[0], wroot_ref[...],
                   preferred_element_type=jnp.float32)
         + b_ref[...])
    t = jnp.maximum(t, 0.0)
    rows = pl.program_id(1) * BS + lax.broadcasted_iota(jnp.int32, (BS, D), 0)
    t = jnp.where(rows < N, t, 0.0)
    xn_ref[...] = t
    yn_ref[...] = jnp.dot(t, wnext_ref[...], preferred_element_type=jnp.float32)


def _tc_mid(agg, x, fms, w_root, b, w_rel_next):
    return pl.pallas_call(
        _tc_mid_body,
        grid=(2, NBR),
        in_specs=[
            pl.BlockSpec((BS, D), lambda v, i: (v * NBR + i, 0)),
            pl.BlockSpec((BS, D), lambda v, i: (v * NBR + i, 0)),
            pl.BlockSpec((1, 1, D), lambda v, i: (v, 0, 0)),
            pl.BlockSpec((D, D), lambda v, i: (0, 0)),
            pl.BlockSpec((1, D), lambda v, i: (0, 0)),
            pl.BlockSpec((D, D), lambda v, i: (0, 0)),
        ],
        out_specs=[
            pl.BlockSpec((BS, D), lambda v, i: (v * NBR + i, 0)),
            pl.BlockSpec((BS, D), lambda v, i: (v * NBR + i, 0)),
        ],
        out_shape=[
            jax.ShapeDtypeStruct((2 * NP, D), jnp.float32),
            jax.ShapeDtypeStruct((2 * NP, D), jnp.float32),
        ],
    )(agg, x, fms.reshape(2, 1, D), w_root, b.reshape(1, D), w_rel_next)


def _tc_last_body(agg_ref, x_ref, wroot_ref, b_ref, wlin_ref, blin_ref, z_ref):
    t = (agg_ref[...]
         + jnp.dot(x_ref[...], wroot_ref[...], preferred_element_type=jnp.float32)
         + b_ref[...])
    t = jnp.maximum(t, 0.0)
    z_ref[...] = (jnp.dot(t, wlin_ref[...], preferred_element_type=jnp.float32)
                  + blin_ref[...])


def _tc_last(agg, x, w_root, b, w_lin, b_lin):
    return pl.pallas_call(
        _tc_last_body,
        grid=(2, NBR),
        in_specs=[
            pl.BlockSpec((BS, D), lambda v, i: (v * NBR + i, 0)),
            pl.BlockSpec((BS, D), lambda v, i: (v * NBR + i, 0)),
            pl.BlockSpec((D, D), lambda v, i: (0, 0)),
            pl.BlockSpec((1, D), lambda v, i: (0, 0)),
            pl.BlockSpec((D, D), lambda v, i: (0, 0)),
            pl.BlockSpec((1, D), lambda v, i: (0, 0)),
        ],
        out_specs=pl.BlockSpec((BS, D), lambda v, i: (v * NBR + i, 0)),
        out_shape=jax.ShapeDtypeStruct((2 * NP, D), jnp.float32),
    )(agg, x, w_root, b.reshape(1, D), w_lin, b_lin.reshape(1, D))


def kernel(x, edge_index, W_rel0, b_rel0, W_root0, W_rel1, b_rel1, W_root1,
           W_rel2, b_rel2, W_root2, W_lin, b_lin):
    src = edge_index[0]
    dst = edge_index[1]
    idx, fm1, fm2 = _prep_edges(src, dst)
    fms = jnp.stack([fm1, fm2])
    ones = jnp.ones_like(fms)
    xp = jnp.zeros((NP, D), jnp.float32).at[:N].set(x)
    x0 = jnp.concatenate([xp, xp], axis=0)
    zrows = jnp.zeros((NP, D), jnp.float32)

    y0 = _tc_first(x0, fms, W_rel0)
    a0 = _sc_agg(y0, idx, zrows)
    x1, y1 = _tc_mid(a0, x0, fms, W_root0, b_rel0, W_rel1)
    a1 = _sc_agg(y1, idx, zrows)
    x2, y2 = _tc_mid(a1, x1, ones, W_root1, b_rel1, W_rel2)
    a2 = _sc_agg(y2, idx, zrows)
    z = _tc_last(a2, x2, W_root2, b_rel2, W_lin, b_lin)
    return (z[:N], z[NP:NP + N])
